# baseline (speedup 1.0000x reference)
import functools

import jax
import jax.numpy as jnp
from jax import lax
from jax.experimental import pallas as pl
from jax.experimental.pallas import tpu as pltpu
from jax.experimental.pallas import tpu_sc as plsc

EMBED_DIMENSION = 300
EMBED_MAX_NORM = 1.0
VOCAB = 100000
BATCH = 1024

N_TILE = 4096

_NC = 2
_NS = 16
_NW = _NC * _NS
_B_PER_W = BATCH // _NW


def _sc_gather(inputs, emb_table):
    mesh = plsc.ScalarSubcoreMesh(axis_name="c", num_cores=_NC)
    b_per_c = BATCH // _NC

    @functools.partial(
        pl.kernel,
        mesh=mesh,
        out_type=jax.ShapeDtypeStruct((BATCH, EMBED_DIMENSION), jnp.float32),
        scratch_types=[
            pltpu.SMEM((b_per_c,), jnp.int32),
            pltpu.SemaphoreType.DMA,
        ],
    )
    def k(idx_hbm, table_hbm, out_hbm, idx_s, sem):
        cid = lax.axis_index("c")
        base = cid * b_per_c
        pltpu.sync_copy(idx_hbm.at[pl.ds(base, b_per_c)], idx_s)

        def body(r, carry):
            row = idx_s[r]
            pltpu.make_async_copy(
                table_hbm.at[pl.ds(row, 1), :],
                out_hbm.at[pl.ds(base + r, 1), :],
                sem,
            ).start()
            return carry

        lax.fori_loop(0, b_per_c, body, 0)

        def drain(r, carry):
            pltpu.make_async_copy(
                table_hbm.at[pl.ds(0, 1), :],
                out_hbm.at[pl.ds(0, 1), :],
                sem,
            ).wait()
            return carry

        lax.fori_loop(0, b_per_c, drain, 0)

    return k(inputs, emb_table)


def _matmul_kernel(emb_ref, w_ref, b_ref, out_ref, ebf_ref):
    @pl.when(pl.program_id(0) == 0)
    def _():
        e = emb_ref[...]
        nrm = jnp.sqrt(jnp.sum(e * e, axis=1, keepdims=True))
        scale = jnp.minimum(1.0, EMBED_MAX_NORM / jnp.maximum(nrm, 1e-7))
        ebf_ref[...] = (e * scale).astype(jnp.bfloat16)

    e = ebf_ref[...]
    w = w_ref[...].astype(jnp.bfloat16)
    acc = jax.lax.dot_general(
        e, w, (((1,), (1,)), ((), ())), preferred_element_type=jnp.float32
    )
    out_ref[...] = acc + b_ref[0, :][None, :]


def _projection(emb, W, b):
    n_blocks = pl.cdiv(VOCAB, N_TILE)
    b2 = b.reshape(1, VOCAB)
    return pl.pallas_call(
        _matmul_kernel,
        grid=(n_blocks,),
        in_specs=[
            pl.BlockSpec((BATCH, EMBED_DIMENSION), lambda j: (0, 0)),
            pl.BlockSpec((N_TILE, EMBED_DIMENSION), lambda j: (j, 0)),
            pl.BlockSpec((1, N_TILE), lambda j: (0, j)),
        ],
        out_specs=pl.BlockSpec((BATCH, N_TILE), lambda j: (0, j)),
        out_shape=jax.ShapeDtypeStruct((BATCH, VOCAB), jnp.float32),
        scratch_shapes=[pltpu.VMEM((BATCH, EMBED_DIMENSION), jnp.bfloat16)],
    )(emb, W, b2)


@jax.jit
def kernel(inputs, emb_table, W, b):
    emb = _sc_gather(inputs, emb_table)
    return _projection(emb, W, b)
